# baseline (speedup 1.0000x reference)
import jax
import jax.numpy as jnp
from jax.experimental import pallas as pl
from jax.experimental.pallas import tpu as pltpu

B, S, D, E = 4, 8192, 2048, 64
CHUNK = 512


def _router_kernel(x_ref, w_ref, b_ref, out_ref, acc_ref):
    i = pl.program_id(0)
    n = pl.num_programs(0)

    @pl.when(i == 0)
    def _init():
        acc_ref[...] = jnp.zeros_like(acc_ref)

    acc_ref[...] += jnp.sum(x_ref[...], axis=1)

    @pl.when(i == n - 1)
    def _fin():
        s = acc_ref[...]
        logits = jax.lax.dot_general(
            s, w_ref[...],
            dimension_numbers=(((1,), (1,)), ((), ())),
            preferred_element_type=jnp.float32,
        ) + b_ref[...]
        out_ref[...] = jnp.argmax(logits, axis=1).astype(jnp.int32)[None, :]


def kernel(x, W, b):
    grid = (S // CHUNK,)
    out = pl.pallas_call(
        _router_kernel,
        grid=grid,
        in_specs=[
            pl.BlockSpec((B, CHUNK, D), lambda i: (0, i, 0)),
            pl.BlockSpec((E, D), lambda i: (0, 0)),
            pl.BlockSpec((1, E), lambda i: (0, 0)),
        ],
        out_specs=pl.BlockSpec((1, B), lambda i: (0, 0)),
        out_shape=jax.ShapeDtypeStruct((1, B), jnp.int32),
        scratch_shapes=[pltpu.VMEM((B, D), jnp.float32)],
    )(x, W, b.reshape(1, E))
    return out.reshape(B)


# speedup vs baseline: 1.0286x; 1.0286x over previous
import jax
import jax.numpy as jnp
from jax.experimental import pallas as pl
from jax.experimental.pallas import tpu as pltpu

B, S, D, E = 4, 8192, 2048, 64
CHUNK = 256


def _router_kernel(x_ref, w_ref, b_ref, out_ref, acc_ref):
    i = pl.program_id(0)
    n = pl.num_programs(0)

    @pl.when(i == 0)
    def _init():
        acc_ref[...] = jnp.zeros_like(acc_ref)

    acc_ref[...] += jnp.sum(x_ref[...], axis=1)

    @pl.when(i == n - 1)
    def _fin():
        s = acc_ref[...]
        logits = jax.lax.dot_general(
            s, w_ref[...],
            dimension_numbers=(((1,), (1,)), ((), ())),
            preferred_element_type=jnp.float32,
        ) + b_ref[...]
        out_ref[...] = jnp.argmax(logits, axis=1).astype(jnp.int32)[None, :]


def kernel(x, W, b):
    grid = (S // CHUNK,)
    out = pl.pallas_call(
        _router_kernel,
        grid=grid,
        in_specs=[
            pl.BlockSpec((B, CHUNK, D), lambda i: (0, i, 0)),
            pl.BlockSpec((E, D), lambda i: (0, 0)),
            pl.BlockSpec((1, E), lambda i: (0, 0)),
        ],
        out_specs=pl.BlockSpec((1, B), lambda i: (0, 0)),
        out_shape=jax.ShapeDtypeStruct((1, B), jnp.int32),
        scratch_shapes=[pltpu.VMEM((B, D), jnp.float32)],
    )(x, W, b.reshape(1, E))
    return out.reshape(B)
